# baseline (speedup 1.0000x reference)
import functools

import jax
import jax.numpy as jnp
from jax import lax
from jax.experimental import pallas as pl
from jax.experimental.pallas import tpu as pltpu
from jax.experimental.pallas import tpu_sc as plsc

NUM_CORES = 2
NUM_SUBCORES = 16
NUM_WORKERS = NUM_CORES * NUM_SUBCORES
LANES = 16


@functools.partial(jax.jit, static_argnums=(2, 3, 4))
def _sc_gather(table, idx, num_layers, dual, row):
    pool = table.shape[0] // (num_layers * dual)
    batch = idx.shape[0]
    n_out = num_layers * batch * dual
    rows_per_w = n_out // NUM_WORKERS
    chunk = LANES
    n_chunks = rows_per_w // chunk
    assert dual * batch == 1 << ((dual * batch).bit_length() - 1)
    assert batch == 1 << (batch.bit_length() - 1)
    db_shift = (dual * batch).bit_length() - 1
    b_shift = batch.bit_length() - 1

    mesh = plsc.VectorSubcoreMesh(core_axis_name="c", subcore_axis_name="s")

    @functools.partial(
        pl.kernel,
        mesh=mesh,
        out_type=jax.ShapeDtypeStruct((n_out, row), jnp.float32),
        scratch_types=[
            pltpu.VMEM((batch,), jnp.int32),
            pltpu.VMEM((chunk,), jnp.int32),
            pltpu.VMEM((chunk, row), jnp.float32),
            pltpu.SemaphoreType.DMA,
        ],
    )
    def k(table_hbm, idx_hbm, out_hbm, idx_v, src_v, rows_v, sem):
        cid = lax.axis_index("c")
        sid = lax.axis_index("s")
        wid = sid * NUM_CORES + cid
        base = wid * rows_per_w
        pltpu.sync_copy(idx_hbm, idx_v)

        def step(i, carry):
            r0 = base + i * chunk
            l0 = lax.shift_right_logical(r0, db_shift)
            rem0 = lax.bitwise_and(r0, (dual * batch) - 1)
            d0 = lax.shift_right_logical(rem0, b_shift)
            b0 = lax.bitwise_and(rem0, batch - 1)
            off = (l0 * dual + d0) * pool
            src_v[...] = off + idx_v[pl.ds(b0, chunk)]
            pltpu.async_copy(table_hbm.at[src_v], rows_v, sem).wait()
            pltpu.sync_copy(rows_v, out_hbm.at[pl.ds(r0, chunk)])
            return carry

        lax.fori_loop(0, n_chunks, step, 0)

    return k(table, idx)


def kernel(prompt, idx):
    num_layers, dual, pool, length, heads, head_dim = prompt.shape
    batch = idx.shape[0]
    row = length * heads * head_dim
    table = prompt.reshape(num_layers * dual * pool, row)
    out = _sc_gather(table, idx.astype(jnp.int32), num_layers, dual, row)
    return out.reshape(num_layers, batch, dual, length, heads, head_dim)


# speedup vs baseline: 1.0059x; 1.0059x over previous
import functools

import jax
import jax.numpy as jnp
from jax import lax
from jax.experimental import pallas as pl
from jax.experimental.pallas import tpu as pltpu
from jax.experimental.pallas import tpu_sc as plsc

NUM_CORES = 2
NUM_SUBCORES = 16
NUM_WORKERS = NUM_CORES * NUM_SUBCORES
LANES = 16


@functools.partial(jax.jit, static_argnums=(2, 3, 4))
def _sc_gather(table, idx, num_layers, dual, row):
    pool = table.shape[0] // (num_layers * dual)
    batch = idx.shape[0]
    n_out = num_layers * batch * dual
    rows_per_w = n_out // NUM_WORKERS
    chunk = LANES
    n_chunks = rows_per_w // chunk
    assert dual * batch == 1 << ((dual * batch).bit_length() - 1)
    assert batch == 1 << (batch.bit_length() - 1)
    db_shift = (dual * batch).bit_length() - 1
    b_shift = batch.bit_length() - 1

    mesh = plsc.VectorSubcoreMesh(core_axis_name="c", subcore_axis_name="s")

    @functools.partial(
        pl.kernel,
        mesh=mesh,
        out_type=jax.ShapeDtypeStruct((n_out, row), jnp.float32),
        scratch_types=[
            pltpu.VMEM((batch,), jnp.int32),
            pltpu.VMEM((chunk,), jnp.int32),
            pltpu.VMEM((chunk,), jnp.int32),
            pltpu.VMEM((chunk, row), jnp.float32),
            pltpu.VMEM((chunk, row), jnp.float32),
            pltpu.SemaphoreType.DMA,
            pltpu.SemaphoreType.DMA,
            pltpu.SemaphoreType.DMA,
            pltpu.SemaphoreType.DMA,
        ],
    )
    def k(table_hbm, idx_hbm, out_hbm, idx_v, si0, si1, b0, b1,
          g0, g1, s0, s1):
        cid = lax.axis_index("c")
        sid = lax.axis_index("s")
        wid = sid * NUM_CORES + cid
        base = wid * rows_per_w
        pltpu.sync_copy(idx_hbm, idx_v)

        si = (si0, si1)
        buf = (b0, b1)
        gsem = (g0, g1)
        ssem = (s0, s1)

        def row0(j):
            return base + j * chunk

        def start_gather(j):
            r0 = row0(j)
            l0 = lax.shift_right_logical(r0, db_shift)
            rem0 = lax.bitwise_and(r0, (dual * batch) - 1)
            d0 = lax.shift_right_logical(rem0, b_shift)
            bb0 = lax.bitwise_and(rem0, batch - 1)
            off = (l0 * dual + d0) * pool
            si[j % 2][...] = off + idx_v[pl.ds(bb0, chunk)]
            return pltpu.async_copy(
                table_hbm.at[si[j % 2]], buf[j % 2], gsem[j % 2]
            )

        def start_store(j):
            return pltpu.async_copy(
                buf[j % 2], out_hbm.at[pl.ds(row0(j), chunk)], ssem[j % 2]
            )

        gathers = {0: start_gather(0)}
        stores = {}
        for j in range(n_chunks):
            if j + 1 < n_chunks:
                if j >= 1:
                    stores[j - 1].wait()
                gathers[j + 1] = start_gather(j + 1)
            gathers[j].wait()
            stores[j] = start_store(j)
        stores[n_chunks - 2].wait()
        stores[n_chunks - 1].wait()

    return k(table, idx)


def kernel(prompt, idx):
    num_layers, dual, pool, length, heads, head_dim = prompt.shape
    batch = idx.shape[0]
    row = length * heads * head_dim
    table = prompt.reshape(num_layers * dual * pool, row)
    out = _sc_gather(table, idx.astype(jnp.int32), num_layers, dual, row)
    return out.reshape(num_layers, batch, dual, length, heads, head_dim)


# speedup vs baseline: 1.3316x; 1.3238x over previous
import functools

import jax
import jax.numpy as jnp
from jax import lax
from jax.experimental import pallas as pl
from jax.experimental.pallas import tpu as pltpu
from jax.experimental.pallas import tpu_sc as plsc

NUM_CORES = 2
NUM_SUBCORES = 16
NUM_WORKERS = NUM_CORES * NUM_SUBCORES
LANES = 16
WIDTH = 128
IDX_PER_DMA = 120


@functools.partial(jax.jit, static_argnums=(2, 3, 4))
def _sc_gather(table, idx, num_layers, dual, sub):
    pool = table.shape[0] // (num_layers * dual * sub)
    batch = idx.shape[0]
    n_out = num_layers * batch * dual
    rows_per_w = n_out // NUM_WORKERS
    chunk = LANES
    n_chunks = rows_per_w // chunk
    n_idx = chunk * sub
    assert n_idx % IDX_PER_DMA == 0
    dmas_per_chunk = n_idx // IDX_PER_DMA
    assert dual * batch == 1 << ((dual * batch).bit_length() - 1)
    assert batch == 1 << (batch.bit_length() - 1)
    db_shift = (dual * batch).bit_length() - 1
    b_shift = batch.bit_length() - 1

    mesh = plsc.VectorSubcoreMesh(core_axis_name="c", subcore_axis_name="s")

    @functools.partial(
        pl.kernel,
        mesh=mesh,
        out_type=jax.ShapeDtypeStruct((n_out * sub, WIDTH), jnp.float32),
        scratch_types=[
            pltpu.VMEM((batch,), jnp.int32),
            pltpu.VMEM((n_idx,), jnp.int32),
            pltpu.VMEM((n_idx,), jnp.int32),
            pltpu.VMEM((n_idx, WIDTH), jnp.float32),
            pltpu.VMEM((n_idx, WIDTH), jnp.float32),
            pltpu.SemaphoreType.DMA,
            pltpu.SemaphoreType.DMA,
            pltpu.SemaphoreType.DMA,
            pltpu.SemaphoreType.DMA,
        ],
    )
    def k(table_hbm, idx_hbm, out_hbm, idx_v, si0, si1, b0, b1,
          g0, g1, s0, s1):
        cid = lax.axis_index("c")
        sid = lax.axis_index("s")
        wid = sid * NUM_CORES + cid
        base = wid * rows_per_w
        pltpu.sync_copy(idx_hbm, idx_v)

        si = (si0, si1)
        buf = (b0, b1)
        gsem = (g0, g1)
        ssem = (s0, s1)
        lanes16 = lax.iota(jnp.int32, LANES)

        def row0(j):
            return base + j * chunk

        def start_gather(j):
            r0 = row0(j)
            l0 = lax.shift_right_logical(r0, db_shift)
            rem0 = lax.bitwise_and(r0, (dual * batch) - 1)
            d0 = lax.shift_right_logical(rem0, b_shift)
            bb0 = lax.bitwise_and(rem0, batch - 1)
            off = (l0 * dual + d0) * pool
            src = (off + idx_v[pl.ds(bb0, chunk)]) * sub
            s_ref = si[j % 2]
            for g in range(n_idx // LANES):
                pos0 = g * LANES
                i_lo = pos0 // sub
                i_hi = (pos0 + LANES - 1) // sub
                t_lo = pos0 - i_lo * sub
                if i_lo == i_hi:
                    vals = src[i_lo] + t_lo + lanes16
                else:
                    nb = i_hi * sub - pos0
                    vals = jnp.where(
                        lanes16 < nb,
                        src[i_lo] + t_lo + lanes16,
                        src[i_hi] - nb + lanes16,
                    )
                s_ref[pl.ds(pos0, LANES)] = vals
            copies = []
            for kdma in range(dmas_per_chunk):
                copies.append(pltpu.async_copy(
                    table_hbm.at[s_ref.at[pl.ds(kdma * IDX_PER_DMA,
                                                IDX_PER_DMA)]],
                    buf[j % 2].at[pl.ds(kdma * IDX_PER_DMA, IDX_PER_DMA)],
                    gsem[j % 2],
                ))
            return copies

        def start_store(j):
            return pltpu.async_copy(
                buf[j % 2],
                out_hbm.at[pl.ds(row0(j) * sub, n_idx)],
                ssem[j % 2],
            )

        gathers = {0: start_gather(0)}
        stores = {}
        for j in range(n_chunks):
            if j + 1 < n_chunks:
                if j >= 1:
                    stores[j - 1].wait()
                gathers[j + 1] = start_gather(j + 1)
            for c in gathers.pop(j):
                c.wait()
            stores[j] = start_store(j)
        stores[n_chunks - 2].wait()
        stores[n_chunks - 1].wait()

    return k(table, idx)


def kernel(prompt, idx):
    num_layers, dual, pool, length, heads, head_dim = prompt.shape
    batch = idx.shape[0]
    row = length * heads * head_dim
    sub = row // WIDTH
    table = prompt.reshape(num_layers * dual * pool * sub, WIDTH)
    out = _sc_gather(table, idx.astype(jnp.int32), num_layers, dual, sub)
    return out.reshape(num_layers, batch, dual, length, heads, head_dim)
